# Initial kernel scaffold; baseline (speedup 1.0000x reference)
import jax
import jax.numpy as jnp
from jax.experimental import pallas as pl


def kernel(fs, stencil_indices, weights):
    raise NotImplementedError("write your pallas kernel here")


# speedup vs baseline: 27.6466x; 27.6466x over previous
import dataclasses
import functools

import jax
import jax.numpy as jnp
from jax import lax
from jax.experimental import pallas as pl
from jax.experimental.pallas import tpu as pltpu
from jax.experimental.pallas import tpu_sc as plsc

N = 100000
M = 32
B = 4
D = 3

NUM_TILES = 32
CHUNK = 64
NODES_PER_TILE = 3136
NCHUNKS = NODES_PER_TILE // CHUNK
IDX_ROWS = CHUNK * M // 128
WCHUNK = CHUNK * D * M


def _sc_body(fs16_hbm, idx_hbm, w_hbm, out_hbm,
             idxbuf, gbuf, wbuf, accbuf, outbuf, sem):
    cid = lax.axis_index("c")
    sid = lax.axis_index("s")
    wid = cid * 16 + sid

    lane = jnp.arange(16, dtype=jnp.int32)
    patt = jnp.minimum(lane & 3, 2) * M

    @pl.loop(0, NCHUNKS)
    def _chunk(i):
        base = pl.multiple_of(
            jnp.minimum(wid * NODES_PER_TILE + i * CHUNK, N - CHUNK), 32)

        idx_row = pl.multiple_of(base * M // 128, 8)
        pltpu.sync_copy(idx_hbm.at[pl.ds(idx_row, IDX_ROWS)], idxbuf)
        copies = [
            pltpu.async_copy(fs16_hbm.at[idxbuf.at[j]],
                             gbuf.at[pl.ds(j * 128, 128)], sem)
            for j in range(IDX_ROWS)
        ]
        pltpu.sync_copy(w_hbm.at[pl.ds(base * D * M, WCHUNK)], wbuf)
        for c in copies:
            c.wait()

        @pl.loop(0, CHUNK)
        def _node(n):
            wbase = n * (D * M)
            acc0 = jnp.zeros((16,), jnp.float32)
            acc1 = jnp.zeros((16,), jnp.float32)
            acc2 = jnp.zeros((16,), jnp.float32)
            acc3 = jnp.zeros((16,), jnp.float32)
            accs = [acc0, acc1, acc2, acc3]
            for m in range(M):
                wv = plsc.load_gather(wbuf, [patt + (wbase + m)])
                gv = gbuf[n * M + m]
                accs[m & 3] = accs[m & 3] + wv * gv
            acc = (accs[0] + accs[1]) + (accs[2] + accs[3])
            accbuf[pl.ds(n * 16, 16)] = acc

        @pl.loop(0, CHUNK // 16)
        def _fold(g):
            rows = (g * 16 + lane) * 16
            for b in range(B):
                s0 = plsc.load_gather(accbuf, [rows + (4 * b + 0)])
                s1 = plsc.load_gather(accbuf, [rows + (4 * b + 1)])
                s2 = plsc.load_gather(accbuf, [rows + (4 * b + 2)])
                s3 = plsc.load_gather(accbuf, [rows + (4 * b + 3)])
                outbuf[pl.ds(b * CHUNK + g * 16, 16)] = (s0 + s1) + (s2 + s3)

        for b in range(B):
            pltpu.sync_copy(outbuf.at[pl.ds(b * CHUNK, CHUNK)],
                            out_hbm.at[pl.ds(b * N + base, CHUNK)])


@jax.jit
def _rbffd_divergence_sc(fs16, idx2d, w_flat):
    mesh = plsc.VectorSubcoreMesh(core_axis_name="c", subcore_axis_name="s")
    cp = pltpu.CompilerParams()
    if "needs_layout_passes" in pltpu.CompilerParams.__dataclass_fields__:
        cp = dataclasses.replace(cp, needs_layout_passes=False)
    if "use_tc_tiling_on_sc" in pltpu.CompilerParams.__dataclass_fields__:
        cp = dataclasses.replace(cp, use_tc_tiling_on_sc=False)
    run = pl.kernel(
        _sc_body,
        out_type=jax.ShapeDtypeStruct((B * N,), jnp.float32),
        mesh=mesh,
        scratch_types=[
            pltpu.VMEM((IDX_ROWS, 128), jnp.int32),
            pltpu.VMEM((CHUNK * M, 16), jnp.float32),
            pltpu.VMEM((WCHUNK,), jnp.float32),
            pltpu.VMEM((CHUNK * 16,), jnp.float32),
            pltpu.VMEM((B * CHUNK,), jnp.float32),
            pltpu.SemaphoreType.DMA,
        ],
        compiler_params=cp,
    )
    return run(fs16, idx2d, w_flat)


def kernel(fs, stencil_indices, weights):
    fs = jnp.asarray(fs, jnp.float32)
    fs16 = jnp.pad(jnp.transpose(fs, (1, 0, 2)),
                   ((0, 0), (0, 0), (0, 1))).reshape(N, 4 * B)
    idx2d = stencil_indices.reshape(N * M // 128, 128)
    w_flat = jnp.asarray(weights, jnp.float32).reshape(-1)
    out_flat = _rbffd_divergence_sc(fs16, idx2d, w_flat)
    return out_flat.reshape(B, N)


# speedup vs baseline: 31.4265x; 1.1367x over previous
import dataclasses
import functools

import jax
import jax.numpy as jnp
from jax import lax
from jax.experimental import pallas as pl
from jax.experimental.pallas import tpu as pltpu
from jax.experimental.pallas import tpu_sc as plsc

N = 100000
M = 32
B = 4
D = 3

NUM_TILES = 32
CHUNK = 64
NODES_PER_TILE = 3136
NCHUNKS = NODES_PER_TILE // CHUNK
IDX_ROWS = CHUNK * M // 128
WCHUNK = CHUNK * D * M


def _sc_body(fs16_hbm, idx_hbm, w_hbm, out_hbm,
             idx0, idx1, g0, g1, w0, w1, accbuf, resbuf, sem0, sem1):
    cid = lax.axis_index("c")
    sid = lax.axis_index("s")
    wid = cid * 16 + sid
    tile_base = jnp.minimum(wid * NODES_PER_TILE, N - NODES_PER_TILE)

    lane = jnp.arange(16, dtype=jnp.int32)
    patt = jnp.minimum(lane & 3, 2) * M

    slots = ((idx0, g0, w0, sem0), (idx1, g1, w1, sem1))

    def fire(i, slot):
        idxb, gb, wb, sem = slots[slot]
        base = pl.multiple_of(tile_base + i * CHUNK, 32)
        pltpu.sync_copy(idx_hbm.at[pl.ds(base * M // 128, IDX_ROWS)], idxb)
        for j in range(IDX_ROWS):
            pltpu.async_copy(fs16_hbm.at[idxb.at[j]],
                             gb.at[pl.ds(j * 128, 128)], sem)
        pltpu.async_copy(w_hbm.at[pl.ds(base * D * M, WCHUNK)], wb, sem)

    def drain(slot):
        idxb, gb, wb, sem = slots[slot]
        pltpu.make_async_copy(fs16_hbm.at[pl.ds(0, CHUNK * M)], gb, sem).wait()
        pltpu.make_async_copy(w_hbm.at[pl.ds(0, WCHUNK)], wb, sem).wait()

    def compute(i, slot):
        _, gb, wb, _ = slots[slot]

        @pl.loop(0, CHUNK)
        def _node(n):
            wbase = n * (D * M)
            acc0 = jnp.zeros((16,), jnp.float32)
            acc1 = jnp.zeros((16,), jnp.float32)
            acc2 = jnp.zeros((16,), jnp.float32)
            acc3 = jnp.zeros((16,), jnp.float32)
            accs = [acc0, acc1, acc2, acc3]
            for m in range(M):
                wv = plsc.load_gather(wb, [patt + (wbase + m)])
                gv = gb[n * M + m]
                accs[m & 3] = accs[m & 3] + wv * gv
            acc = (accs[0] + accs[1]) + (accs[2] + accs[3])
            accbuf[pl.ds(n * 16, 16)] = acc

        @pl.loop(0, CHUNK // 16)
        def _fold(g):
            rows = (g * 16 + lane) * 16
            for b in range(B):
                s0 = plsc.load_gather(accbuf, [rows + (4 * b + 0)])
                s1 = plsc.load_gather(accbuf, [rows + (4 * b + 1)])
                s2 = plsc.load_gather(accbuf, [rows + (4 * b + 2)])
                s3 = plsc.load_gather(accbuf, [rows + (4 * b + 3)])
                resbuf[pl.ds(b * NODES_PER_TILE + i * CHUNK + g * 16, 16)] = (
                    (s0 + s1) + (s2 + s3))

    fire(0, 0)

    @pl.loop(0, NCHUNKS - 1, step=2)
    def _pair(g):
        fire(g + 1, 1)
        drain(0)
        compute(g, 0)
        fire(g + 2, 0)
        drain(1)
        compute(g + 1, 1)

    drain(0)
    compute(NCHUNKS - 1, 0)

    for b in range(B):
        pltpu.sync_copy(
            resbuf.at[pl.ds(b * NODES_PER_TILE, NODES_PER_TILE)],
            out_hbm.at[pl.ds(b * N + tile_base, NODES_PER_TILE)])


@jax.jit
def _rbffd_divergence_sc(fs16, idx2d, w_flat):
    mesh = plsc.VectorSubcoreMesh(core_axis_name="c", subcore_axis_name="s")
    cp = pltpu.CompilerParams()
    if "needs_layout_passes" in pltpu.CompilerParams.__dataclass_fields__:
        cp = dataclasses.replace(cp, needs_layout_passes=False)
    if "use_tc_tiling_on_sc" in pltpu.CompilerParams.__dataclass_fields__:
        cp = dataclasses.replace(cp, use_tc_tiling_on_sc=False)
    run = pl.kernel(
        _sc_body,
        out_type=jax.ShapeDtypeStruct((B * N,), jnp.float32),
        mesh=mesh,
        scratch_types=[
            pltpu.VMEM((IDX_ROWS, 128), jnp.int32),
            pltpu.VMEM((IDX_ROWS, 128), jnp.int32),
            pltpu.VMEM((CHUNK * M, 16), jnp.float32),
            pltpu.VMEM((CHUNK * M, 16), jnp.float32),
            pltpu.VMEM((WCHUNK,), jnp.float32),
            pltpu.VMEM((WCHUNK,), jnp.float32),
            pltpu.VMEM((CHUNK * 16,), jnp.float32),
            pltpu.VMEM((B * NODES_PER_TILE,), jnp.float32),
            pltpu.SemaphoreType.DMA,
            pltpu.SemaphoreType.DMA,
        ],
        compiler_params=cp,
    )
    return run(fs16, idx2d, w_flat)


def kernel(fs, stencil_indices, weights):
    fs = jnp.asarray(fs, jnp.float32)
    fs16 = jnp.pad(jnp.transpose(fs, (1, 0, 2)),
                   ((0, 0), (0, 0), (0, 1))).reshape(N, 4 * B)
    idx2d = stencil_indices.reshape(N * M // 128, 128)
    w_flat = jnp.asarray(weights, jnp.float32).reshape(-1)
    out_flat = _rbffd_divergence_sc(fs16, idx2d, w_flat)
    return out_flat.reshape(B, N)
